# baseline (speedup 1.0000x reference)
import functools
import jax
import jax.numpy as jnp
from jax import lax
from jax.experimental import pallas as pl
from jax.experimental.pallas import tpu as pltpu
from jax.experimental.pallas import tpu_sc as plsc

_ALPHA = 4
_NUM_CORES = 2
_NUM_SUBCORES = 16


def _make_sc_gather(lead, T, S, row):
    n_rows = lead * S
    n_workers = _NUM_CORES * _NUM_SUBCORES
    per_w = -(-n_rows // n_workers)
    mesh = plsc.VectorSubcoreMesh(core_axis_name="c", subcore_axis_name="s")

    @functools.partial(
        pl.kernel,
        mesh=mesh,
        out_type=jax.ShapeDtypeStruct((n_rows, row), jnp.float32),
        scratch_types=[pltpu.SemaphoreType.DMA],
    )
    def sc_gather(x_hbm, out_hbm, sem):
        wid = lax.axis_index("s") * _NUM_CORES + lax.axis_index("c")
        base = wid * per_w
        copies = []
        for j in range(per_w):
            r = base + j
            i = r // S
            k = r - i * S
            src = i * T + ((T - 1) * k) // (S - 1)
            copies.append(pltpu.async_copy(x_hbm.at[src], out_hbm.at[r], sem))
        for c in copies:
            c.wait()

    return sc_gather


def kernel(frames):
    temporal_axis = 1 if frames.ndim == 4 else 2
    T = frames.shape[temporal_axis]
    S = T // _ALPHA

    if frames.ndim == 4:
        C, _, H, W = frames.shape
        lead = C
    else:
        B, C, _, H, W = frames.shape
        lead = B * C

    row = H * W
    x = frames.reshape(lead * T, row)
    slow = _make_sc_gather(lead, T, S, row)(x)

    if frames.ndim == 4:
        slow = slow.reshape(C, S, H, W)
    else:
        slow = slow.reshape(B, C, S, H, W)
    return (slow, frames)
